# baseline (speedup 1.0000x reference)
import jax
import jax.numpy as jnp
from jax.experimental import pallas as pl
from jax.experimental.pallas import tpu as pltpu

TOKENS = 8192
D_MODEL = 2048
HIDDEN = 4096
EXPERTS = 16

TB = 1024
HB = 512
T = TOKENS // TB
H = HIDDEN // HB

_BF = jnp.bfloat16
_DN = (((1,), (1,)), ((), ()))


def _wn_kernel(w1v_ref, w1g_ref, n1_ref, w2v_ref, w2g_ref, n2_ref,
               w12_ref):
    w1n = (w1v_ref[...] * w1g_ref[...][:, None]
           / (n1_ref[...][:, None] + 1e-12))
    w12_ref[:HB, :] = w1n.astype(_BF)
    w2n = (w2v_ref[...] * w2g_ref[...][:, None]
           / (n2_ref[...][:, None] + 1e-12))
    w12_ref[HB:, :] = w2n.astype(_BF)


def _ln_kernel(x_ref, mu_ref, var_ref, ln_g_ref, ln_b_ref, xn_ref):
    xb = x_ref[...]
    mu = mu_ref[...][:, None]
    var = var_ref[...][:, None]
    xn = (xb - mu) / jnp.sqrt(var + 1e-5)
    xn = xn * ln_g_ref[...][None, :] + ln_b_ref[...][None, :]
    xn_ref[...] = xn.astype(_BF)


def _swiglu(hh, b1_ref, b2_ref, sw_ref, hidx):
    sl = pl.ds(hidx * HB, HB)
    h1 = hh[:, :HB] + b1_ref[sl][None, :]
    h2 = hh[:, HB:] + b2_ref[sl][None, :]
    sw_ref[:, sl] = (h1 * jax.nn.sigmoid(h2)).astype(_BF)


def _router_kernel(xn_ref, w12_ref, b1_ref, b2_ref,
                   w3v_ref, w3g_ref, n3_ref, b3_ref, eb_ref,
                   idx_ref, prob_ref, usage_ref,
                   sw_ref, hh_ref, w3n_ref):
    t = pl.program_id(0)
    h = pl.program_id(1)

    @pl.when(jnp.logical_and(t == 0, h == 0))
    def _prologue():
        v3 = w3v_ref[...]
        w3n = v3 * w3g_ref[...][:, None] / (n3_ref[...][:, None] + 1e-12)
        w3n_ref[...] = w3n.astype(_BF)

    @pl.when(h > 0)
    def _epilogue_prev():
        _swiglu(hh_ref[...], b1_ref, b2_ref, sw_ref, h - 1)

    hh_ref[...] = jax.lax.dot_general(xn_ref[...], w12_ref[...], _DN,
                                      preferred_element_type=jnp.float32)

    @pl.when(h == H - 1)
    def _route():
        _swiglu(hh_ref[...], b1_ref, b2_ref, sw_ref, h)
        logits = jax.lax.dot_general(sw_ref[...], w3n_ref[...], _DN,
                                     preferred_element_type=jnp.float32)
        logits = logits + b3_ref[...][None, :] + eb_ref[...][None, :]
        cols = jax.lax.broadcasted_iota(jnp.int32, logits.shape, 1)
        m1 = jnp.max(logits, axis=1, keepdims=True)
        i1 = jnp.min(jnp.where(logits == m1, cols, EXPERTS),
                     axis=1, keepdims=True)
        masked = jnp.where(cols == i1, -jnp.inf, logits)
        m2 = jnp.max(masked, axis=1, keepdims=True)
        i2 = jnp.min(jnp.where(masked == m2, cols, EXPERTS),
                     axis=1, keepdims=True)
        d = m1 - m2
        idx_ref[...] = jnp.concatenate([i1, i2], axis=1)
        prob_ref[...] = jnp.concatenate(
            [jax.nn.sigmoid(d), jax.nn.sigmoid(-d)], axis=1)
        one = ((cols == i1) | (cols == i2)).astype(jnp.float32)
        part = jnp.sum(one, axis=0, keepdims=True)

        @pl.when(t == 0)
        def _init_usage():
            usage_ref[...] = part

        @pl.when(t != 0)
        def _acc_usage():
            usage_ref[...] += part


def kernel(x, ln_g, ln_b, w1_v, w1_g, b1, w2_v, w2_g, b2,
           w3_v, w3_g, b3, expert_bias):
    mu = jnp.mean(x, axis=-1, keepdims=True)
    var = jnp.mean((x - mu) ** 2, axis=-1, keepdims=True)
    n1 = jnp.sqrt(jnp.sum(w1_v * w1_v, axis=1, keepdims=True) + 0.0)
    n2 = jnp.sqrt(jnp.sum(w2_v * w2_v, axis=1, keepdims=True) + 0.0)
    n3 = jnp.sqrt(jnp.sum(w3_v * w3_v, axis=1, keepdims=True) + 0.0)

    w12 = pl.pallas_call(
        _wn_kernel,
        grid=(H,),
        in_specs=[
            pl.BlockSpec((HB, D_MODEL), lambda i: (i, 0)),
            pl.BlockSpec((HB,), lambda i: (i,)),
            pl.BlockSpec((HB,), lambda i: (i,)),
            pl.BlockSpec((HB, D_MODEL), lambda i: (i, 0)),
            pl.BlockSpec((HB,), lambda i: (i,)),
            pl.BlockSpec((HB,), lambda i: (i,)),
        ],
        out_specs=pl.BlockSpec((2 * HB, D_MODEL), lambda i: (i, 0)),
        out_shape=jax.ShapeDtypeStruct((2 * HIDDEN, D_MODEL), _BF),
    )(w1_v, w1_g, n1.reshape(HIDDEN), w2_v, w2_g, n2.reshape(HIDDEN))

    xn = pl.pallas_call(
        _ln_kernel,
        grid=(T,),
        in_specs=[
            pl.BlockSpec((TB, D_MODEL), lambda i: (i, 0)),
            pl.BlockSpec((TB,), lambda i: (i,)),
            pl.BlockSpec((TB,), lambda i: (i,)),
            pl.BlockSpec((D_MODEL,), lambda i: (0,)),
            pl.BlockSpec((D_MODEL,), lambda i: (0,)),
        ],
        out_specs=pl.BlockSpec((TB, D_MODEL), lambda i: (i, 0)),
        out_shape=jax.ShapeDtypeStruct((TOKENS, D_MODEL), _BF),
    )(x, mu.reshape(TOKENS), var.reshape(TOKENS), ln_g, ln_b)

    out_shapes = (
        jax.ShapeDtypeStruct((TOKENS, 2), jnp.int32),
        jax.ShapeDtypeStruct((TOKENS, 2), jnp.float32),
        jax.ShapeDtypeStruct((1, EXPERTS), jnp.float32),
    )
    grid = (T, H)
    in_specs = [
        pl.BlockSpec((TB, D_MODEL), lambda t, h: (t, 0)),
        pl.BlockSpec((2 * HB, D_MODEL), lambda t, h: (h, 0)),
        pl.BlockSpec((HIDDEN,), lambda t, h: (0,)),
        pl.BlockSpec((HIDDEN,), lambda t, h: (0,)),
        pl.BlockSpec((EXPERTS, HIDDEN), lambda t, h: (0, 0)),
        pl.BlockSpec((EXPERTS,), lambda t, h: (0,)),
        pl.BlockSpec((EXPERTS,), lambda t, h: (0,)),
        pl.BlockSpec((EXPERTS,), lambda t, h: (0,)),
        pl.BlockSpec((EXPERTS,), lambda t, h: (0,)),
    ]
    out_specs = (
        pl.BlockSpec((TB, 2), lambda t, h: (t, 0)),
        pl.BlockSpec((TB, 2), lambda t, h: (t, 0)),
        pl.BlockSpec((1, EXPERTS), lambda t, h: (0, 0)),
    )
    scratch_shapes = [
        pltpu.VMEM((TB, HIDDEN), _BF),
        pltpu.VMEM((TB, 2 * HB), jnp.float32),
        pltpu.VMEM((EXPERTS, HIDDEN), _BF),
    ]
    idx, probs, usage = pl.pallas_call(
        _router_kernel,
        grid=grid,
        in_specs=in_specs,
        out_specs=out_specs,
        out_shape=out_shapes,
        scratch_shapes=scratch_shapes,
        compiler_params=pltpu.CompilerParams(
            dimension_semantics=("arbitrary", "arbitrary"),
        ),
    )(xn, w12, b1, b2, w3_v, w3_g, n3.reshape(EXPERTS), b3, expert_bias)
    return (idx, probs, usage.reshape(EXPERTS))


# speedup vs baseline: 1.0616x; 1.0616x over previous
import jax
import jax.numpy as jnp
from jax.experimental import pallas as pl
from jax.experimental.pallas import tpu as pltpu

TOKENS = 8192
D_MODEL = 2048
HIDDEN = 4096
EXPERTS = 16

TB = 2048
HB = 512
T = TOKENS // TB
H = HIDDEN // HB

_BF = jnp.bfloat16
_DN = (((1,), (1,)), ((), ()))


def _wn_kernel(w1v_ref, w1g_ref, n1_ref, w2v_ref, w2g_ref, n2_ref,
               w12_ref):
    w1n = (w1v_ref[...] * w1g_ref[...][:, None]
           / (n1_ref[...][:, None] + 1e-12))
    w12_ref[:HB, :] = w1n.astype(_BF)
    w2n = (w2v_ref[...] * w2g_ref[...][:, None]
           / (n2_ref[...][:, None] + 1e-12))
    w12_ref[HB:, :] = w2n.astype(_BF)


def _ln_kernel(x_ref, mu_ref, var_ref, ln_g_ref, ln_b_ref, xn_ref):
    xb = x_ref[...]
    mu = mu_ref[...][:, None]
    var = var_ref[...][:, None]
    xn = (xb - mu) / jnp.sqrt(var + 1e-5)
    xn = xn * ln_g_ref[...][None, :] + ln_b_ref[...][None, :]
    xn_ref[...] = xn.astype(_BF)


def _swiglu(hh, b1_ref, b2_ref, sw_ref, hidx):
    sl = pl.ds(hidx * HB, HB)
    h1 = hh[:, :HB] + b1_ref[sl][None, :]
    h2 = hh[:, HB:] + b2_ref[sl][None, :]
    sw_ref[:, sl] = (h1 * jax.nn.sigmoid(h2)).astype(_BF)


def _router_kernel(xn_ref, w12_ref, b1_ref, b2_ref,
                   w3v_ref, w3g_ref, n3_ref, b3_ref, eb_ref,
                   idx_ref, prob_ref, usage_ref,
                   sw_ref, hhb_ref, w3n_ref):
    t = pl.program_id(0)
    j = pl.program_id(1)

    @pl.when(jnp.logical_and(t == 0, j == 0))
    def _prologue():
        v3 = w3v_ref[...]
        w3n = v3 * w3g_ref[...][:, None] / (n3_ref[...][:, None] + 1e-12)
        w3n_ref[...] = w3n.astype(_BF)

    xn = xn_ref[...]
    prev = jnp.where(j == 0, H - 1, 2 * j - 1)
    _swiglu(hhb_ref[...], b1_ref, b2_ref, sw_ref, prev)

    va = jax.lax.dot_general(xn, w12_ref[:2 * HB, :], _DN,
                             preferred_element_type=jnp.float32)
    _swiglu(va, b1_ref, b2_ref, sw_ref, 2 * j)

    hhb_ref[...] = jax.lax.dot_general(xn, w12_ref[2 * HB:, :], _DN,
                                       preferred_element_type=jnp.float32)

    @pl.when(j == H // 2 - 1)
    def _route():
        _swiglu(hhb_ref[...], b1_ref, b2_ref, sw_ref, H - 1)
        logits = jax.lax.dot_general(sw_ref[...], w3n_ref[...], _DN,
                                     preferred_element_type=jnp.float32)
        logits = logits + b3_ref[...][None, :] + eb_ref[...][None, :]
        cols = jax.lax.broadcasted_iota(jnp.int32, logits.shape, 1)
        m1 = jnp.max(logits, axis=1, keepdims=True)
        i1 = jnp.min(jnp.where(logits == m1, cols, EXPERTS),
                     axis=1, keepdims=True)
        masked = jnp.where(cols == i1, -jnp.inf, logits)
        m2 = jnp.max(masked, axis=1, keepdims=True)
        i2 = jnp.min(jnp.where(masked == m2, cols, EXPERTS),
                     axis=1, keepdims=True)
        d = m1 - m2
        idx_ref[...] = jnp.concatenate([i1, i2], axis=1)
        prob_ref[...] = jnp.concatenate(
            [jax.nn.sigmoid(d), jax.nn.sigmoid(-d)], axis=1)
        one = ((cols == i1) | (cols == i2)).astype(jnp.float32)
        part = jnp.sum(one, axis=0, keepdims=True)

        @pl.when(t == 0)
        def _init_usage():
            usage_ref[...] = part

        @pl.when(t != 0)
        def _acc_usage():
            usage_ref[...] += part


def kernel(x, ln_g, ln_b, w1_v, w1_g, b1, w2_v, w2_g, b2,
           w3_v, w3_g, b3, expert_bias):
    mu = jnp.mean(x, axis=-1, keepdims=True)
    var = jnp.mean((x - mu) ** 2, axis=-1, keepdims=True)
    n1 = jnp.sqrt(jnp.sum(w1_v * w1_v, axis=1, keepdims=True) + 0.0)
    n2 = jnp.sqrt(jnp.sum(w2_v * w2_v, axis=1, keepdims=True) + 0.0)
    n3 = jnp.sqrt(jnp.sum(w3_v * w3_v, axis=1, keepdims=True) + 0.0)

    w12 = pl.pallas_call(
        _wn_kernel,
        grid=(H,),
        in_specs=[
            pl.BlockSpec((HB, D_MODEL), lambda i: (i, 0)),
            pl.BlockSpec((HB,), lambda i: (i,)),
            pl.BlockSpec((HB,), lambda i: (i,)),
            pl.BlockSpec((HB, D_MODEL), lambda i: (i, 0)),
            pl.BlockSpec((HB,), lambda i: (i,)),
            pl.BlockSpec((HB,), lambda i: (i,)),
        ],
        out_specs=pl.BlockSpec((2 * HB, D_MODEL), lambda i: (i, 0)),
        out_shape=jax.ShapeDtypeStruct((2 * HIDDEN, D_MODEL), _BF),
    )(w1_v, w1_g, n1.reshape(HIDDEN), w2_v, w2_g, n2.reshape(HIDDEN))

    LB = 1024
    xn = pl.pallas_call(
        _ln_kernel,
        grid=(TOKENS // LB,),
        in_specs=[
            pl.BlockSpec((LB, D_MODEL), lambda i: (i, 0)),
            pl.BlockSpec((LB,), lambda i: (i,)),
            pl.BlockSpec((LB,), lambda i: (i,)),
            pl.BlockSpec((D_MODEL,), lambda i: (0,)),
            pl.BlockSpec((D_MODEL,), lambda i: (0,)),
        ],
        out_specs=pl.BlockSpec((LB, D_MODEL), lambda i: (i, 0)),
        out_shape=jax.ShapeDtypeStruct((TOKENS, D_MODEL), _BF),
    )(x, mu.reshape(TOKENS), var.reshape(TOKENS), ln_g, ln_b)

    out_shapes = (
        jax.ShapeDtypeStruct((TOKENS, 2), jnp.int32),
        jax.ShapeDtypeStruct((TOKENS, 2), jnp.float32),
        jax.ShapeDtypeStruct((1, EXPERTS), jnp.float32),
    )
    grid = (T, H // 2)
    in_specs = [
        pl.BlockSpec((TB, D_MODEL), lambda t, h: (t, 0)),
        pl.BlockSpec((4 * HB, D_MODEL), lambda t, h: (h, 0)),
        pl.BlockSpec((HIDDEN,), lambda t, h: (0,)),
        pl.BlockSpec((HIDDEN,), lambda t, h: (0,)),
        pl.BlockSpec((EXPERTS, HIDDEN), lambda t, h: (0, 0)),
        pl.BlockSpec((EXPERTS,), lambda t, h: (0,)),
        pl.BlockSpec((EXPERTS,), lambda t, h: (0,)),
        pl.BlockSpec((EXPERTS,), lambda t, h: (0,)),
        pl.BlockSpec((EXPERTS,), lambda t, h: (0,)),
    ]
    out_specs = (
        pl.BlockSpec((TB, 2), lambda t, h: (t, 0)),
        pl.BlockSpec((TB, 2), lambda t, h: (t, 0)),
        pl.BlockSpec((1, EXPERTS), lambda t, h: (0, 0)),
    )
    scratch_shapes = [
        pltpu.VMEM((TB, HIDDEN), _BF),
        pltpu.VMEM((TB, 2 * HB), jnp.float32),
        pltpu.VMEM((EXPERTS, HIDDEN), _BF),
    ]
    idx, probs, usage = pl.pallas_call(
        _router_kernel,
        grid=grid,
        in_specs=in_specs,
        out_specs=out_specs,
        out_shape=out_shapes,
        scratch_shapes=scratch_shapes,
        compiler_params=pltpu.CompilerParams(
            dimension_semantics=("arbitrary", "arbitrary"),
        ),
    )(xn, w12, b1, b2, w3_v, w3_g, n3.reshape(EXPERTS), b3, expert_bias)
    return (idx, probs, usage.reshape(EXPERTS))


# speedup vs baseline: 1.0947x; 1.0312x over previous
import jax
import jax.numpy as jnp
from jax.experimental import pallas as pl
from jax.experimental.pallas import tpu as pltpu

TOKENS = 8192
D_MODEL = 2048
HIDDEN = 4096
EXPERTS = 16

TB = 1024
HB = 256
T = TOKENS // TB
H = HIDDEN // HB

_BF = jnp.bfloat16
_DN = (((1,), (1,)), ((), ()))


def _router_kernel(x_ref, mu_ref, var_ref, ln_g_ref, ln_b_ref,
                   w1v_ref, w1g_ref, n1_ref, b1_ref,
                   w2v_ref, w2g_ref, n2_ref, b2_ref,
                   w3v_ref, w3g_ref, n3_ref, b3_ref, eb_ref,
                   idx_ref, prob_ref, usage_ref,
                   xn_ref, sw_ref, w3n_ref):
    t = pl.program_id(0)
    h = pl.program_id(1)

    @pl.when(h == 0)
    def _prologue():
        xb = x_ref[...]
        mu = mu_ref[...][:, None]
        var = var_ref[...][:, None]
        xn = (xb - mu) / jnp.sqrt(var + 1e-5)
        xn = xn * ln_g_ref[...][None, :] + ln_b_ref[...][None, :]
        xn_ref[...] = xn.astype(_BF)
        v3 = w3v_ref[...]
        w3n = v3 * w3g_ref[...][:, None] / (n3_ref[...][:, None] + 1e-12)
        w3n_ref[...] = w3n.astype(_BF)

    xn = xn_ref[...]
    v1 = w1v_ref[...]
    v2 = w2v_ref[...]
    w1n = v1 * w1g_ref[...][:, None] / (n1_ref[...][:, None] + 1e-12)
    w2n = v2 * w2g_ref[...][:, None] / (n2_ref[...][:, None] + 1e-12)

    h1 = jax.lax.dot_general(xn, w1n.astype(_BF), _DN,
                             preferred_element_type=jnp.float32)
    h1 = h1 + b1_ref[...][None, :]
    h2 = jax.lax.dot_general(xn, w2n.astype(_BF), _DN,
                             preferred_element_type=jnp.float32)
    h2 = h2 + b2_ref[...][None, :]
    sw_ref[:, pl.ds(h * HB, HB)] = (h1 * jax.nn.sigmoid(h2)).astype(_BF)

    @pl.when(h == H - 1)
    def _route():
        logits = jax.lax.dot_general(sw_ref[...], w3n_ref[...], _DN,
                                     preferred_element_type=jnp.float32)
        logits = logits + b3_ref[...][None, :] + eb_ref[...][None, :]
        cols = jax.lax.broadcasted_iota(jnp.int32, logits.shape, 1)
        m1 = jnp.max(logits, axis=1, keepdims=True)
        i1 = jnp.min(jnp.where(logits == m1, cols, EXPERTS),
                     axis=1, keepdims=True)
        masked = jnp.where(cols == i1, -jnp.inf, logits)
        m2 = jnp.max(masked, axis=1, keepdims=True)
        i2 = jnp.min(jnp.where(masked == m2, cols, EXPERTS),
                     axis=1, keepdims=True)
        d = m1 - m2
        idx_ref[...] = jnp.concatenate([i1, i2], axis=1)
        prob_ref[...] = jnp.concatenate(
            [jax.nn.sigmoid(d), jax.nn.sigmoid(-d)], axis=1)
        one = ((cols == i1) | (cols == i2)).astype(jnp.float32)
        part = jnp.sum(one, axis=0, keepdims=True)

        @pl.when(t == 0)
        def _init_usage():
            usage_ref[...] = part

        @pl.when(t != 0)
        def _acc_usage():
            usage_ref[...] += part


def kernel(x, ln_g, ln_b, w1_v, w1_g, b1, w2_v, w2_g, b2,
           w3_v, w3_g, b3, expert_bias):
    mu = jnp.mean(x, axis=-1, keepdims=True)
    var = jnp.mean((x - mu) ** 2, axis=-1, keepdims=True)
    n1 = jnp.sqrt(jnp.sum(w1_v * w1_v, axis=1, keepdims=True) + 0.0)
    n2 = jnp.sqrt(jnp.sum(w2_v * w2_v, axis=1, keepdims=True) + 0.0)
    n3 = jnp.sqrt(jnp.sum(w3_v * w3_v, axis=1, keepdims=True) + 0.0)

    out_shapes = (
        jax.ShapeDtypeStruct((TOKENS, 2), jnp.int32),
        jax.ShapeDtypeStruct((TOKENS, 2), jnp.float32),
        jax.ShapeDtypeStruct((1, EXPERTS), jnp.float32),
    )
    grid = (T, H)
    in_specs = [
        pl.BlockSpec((TB, D_MODEL), lambda t, h: (t, 0)),
        pl.BlockSpec((TB,), lambda t, h: (t,)),
        pl.BlockSpec((TB,), lambda t, h: (t,)),
        pl.BlockSpec((D_MODEL,), lambda t, h: (0,)),
        pl.BlockSpec((D_MODEL,), lambda t, h: (0,)),
        pl.BlockSpec((HB, D_MODEL), lambda t, h: (h, 0)),
        pl.BlockSpec((HB,), lambda t, h: (h,)),
        pl.BlockSpec((HB,), lambda t, h: (h,)),
        pl.BlockSpec((HB,), lambda t, h: (h,)),
        pl.BlockSpec((HB, D_MODEL), lambda t, h: (h, 0)),
        pl.BlockSpec((HB,), lambda t, h: (h,)),
        pl.BlockSpec((HB,), lambda t, h: (h,)),
        pl.BlockSpec((HB,), lambda t, h: (h,)),
        pl.BlockSpec((EXPERTS, HIDDEN), lambda t, h: (0, 0)),
        pl.BlockSpec((EXPERTS,), lambda t, h: (0,)),
        pl.BlockSpec((EXPERTS,), lambda t, h: (0,)),
        pl.BlockSpec((EXPERTS,), lambda t, h: (0,)),
        pl.BlockSpec((EXPERTS,), lambda t, h: (0,)),
    ]
    out_specs = (
        pl.BlockSpec((TB, 2), lambda t, h: (t, 0)),
        pl.BlockSpec((TB, 2), lambda t, h: (t, 0)),
        pl.BlockSpec((1, EXPERTS), lambda t, h: (0, 0)),
    )
    scratch_shapes = [
        pltpu.VMEM((TB, D_MODEL), _BF),
        pltpu.VMEM((TB, HIDDEN), _BF),
        pltpu.VMEM((EXPERTS, HIDDEN), _BF),
    ]
    idx, probs, usage = pl.pallas_call(
        _router_kernel,
        grid=grid,
        in_specs=in_specs,
        out_specs=out_specs,
        out_shape=out_shapes,
        scratch_shapes=scratch_shapes,
        compiler_params=pltpu.CompilerParams(
            dimension_semantics=("arbitrary", "arbitrary"),
        ),
    )(x, mu.reshape(TOKENS), var.reshape(TOKENS), ln_g, ln_b,
      w1_v, w1_g, n1.reshape(HIDDEN), b1,
      w2_v, w2_g, n2.reshape(HIDDEN), b2,
      w3_v, w3_g, n3.reshape(EXPERTS), b3, expert_bias)
    return (idx, probs, usage.reshape(EXPERTS))
